# baseline (speedup 1.0000x reference)
import jax
import jax.numpy as jnp
from jax.experimental import pallas as pl

_E = 64
_K = 8
_BT = 2048

_F16_SUBNORM = 6.103515625e-05
_TWO24 = 16777216.0
_INV_TWO24 = 5.960464477539063e-08
_MAGIC = 12582912.0
_MAGIC2 = 12582912.0 - 114688.0
_RANK_MIN_NORMAL = 115712


def _decode_f16(bits16):
    h = bits16.astype(jnp.int32)
    t = h & 0x7FFF
    sb = (h & 0x8000) << 16
    is_sub = t < 1024
    f_n = jax.lax.bitcast_convert_type(t << 13, jnp.float32)
    mult = jax.lax.bitcast_convert_type(
        sb | jnp.where(is_sub, 0x33800000, 0x77800000), jnp.float32)
    f = jnp.where(is_sub, t.astype(jnp.float32), f_n)
    return f * mult


def _round_f16_normal(v):
    b = jax.lax.bitcast_convert_type(v, jnp.int32)
    rb = (b + 0x0FFF + ((b >> 13) & 1)) & ~0x1FFF
    return jax.lax.bitcast_convert_type(rb, jnp.float32)


def _score_keys(q, neg_lane):
    b = jax.lax.bitcast_convert_type(q, jnp.int32)
    kn = (b + 0x0FFF + ((b >> 13) & 1)) >> 13
    ks = ((q * _TWO24 + _MAGIC) - _MAGIC2).astype(jnp.int32)
    kv = jnp.where(q < _F16_SUBNORM, ks, kn)
    return ((kv << 6) | neg_lane).astype(jnp.float32)


def _gating_kernel(x_ref, w_ref, b_ref, vals_ref, idx_ref):
    xb = _decode_f16(x_ref[...]).astype(jnp.bfloat16)
    logits32 = jax.lax.dot_general(
        xb, w_ref[...], (((1,), (1,)), ((), ())),
        preferred_element_type=jnp.float32
    )

    lm = _round_f16_normal(logits32 + b_ref[0:1, :])
    m = jnp.max(lm, axis=-1, keepdims=True)
    e = jnp.exp(lm - m)
    q = e / jnp.sum(e, axis=-1, keepdims=True)

    lane = jax.lax.broadcasted_iota(jnp.int32, q.shape, 1)
    keys = _score_keys(q, _E - 1 - lane)

    tops = []
    for _ in range(_K):
        mk = jnp.max(keys, axis=-1, keepdims=True)
        tops.append(mk)
        keys = jnp.where(keys == mk, -1.0, keys)

    k8 = jnp.concatenate(tops, axis=-1).astype(jnp.int32)
    i = (_E - 1) - (k8 & (_E - 1))
    kv = k8 >> 6
    v_norm = jax.lax.bitcast_convert_type(kv << 13, jnp.float32)
    v_sub = (kv - 114688).astype(jnp.float32) * _INV_TWO24
    v = jnp.where(kv < _RANK_MIN_NORMAL, v_sub, v_norm)

    e2 = jnp.exp(v - v[:, 0:1])
    out = e2 / jnp.sum(e2, axis=-1, keepdims=True)
    vals_ref[...] = out
    idx_ref[...] = i


@jax.jit
def kernel(x, W, b):
    T, D = x.shape
    E = W.shape[0]
    xb = jax.lax.bitcast_convert_type(x, jnp.int16)
    wb = W.astype(jnp.bfloat16)
    b2 = jnp.broadcast_to(b.astype(jnp.float32).reshape(1, E), (8, E))
    grid = (T // _BT,)
    vals, idx = pl.pallas_call(
        _gating_kernel,
        grid=grid,
        in_specs=[
            pl.BlockSpec((_BT, D), lambda t: (t, 0)),
            pl.BlockSpec((E, D), lambda t: (0, 0)),
            pl.BlockSpec((8, E), lambda t: (0, 0)),
        ],
        out_specs=[
            pl.BlockSpec((_BT, _K), lambda t: (t, 0)),
            pl.BlockSpec((_BT, _K), lambda t: (t, 0)),
        ],
        out_shape=[
            jax.ShapeDtypeStruct((T, _K), jnp.float32),
            jax.ShapeDtypeStruct((T, _K), jnp.int32),
        ],
    )(xb, wb, b2)
    return vals.astype(jnp.float16), idx
